# baseline (speedup 1.0000x reference)
import functools

import jax
import jax.numpy as jnp
from jax import lax
from jax.experimental import pallas as pl
from jax.experimental.pallas import tpu as pltpu
from jax.experimental.pallas import tpu_sc as plsc

N = 10000
E = 320000
H = 128
DE = 4
L = 4

NC = 2
NS = 16
NW = NC * NS
EH = E // 2
PT = EH // NW
PTP = 5120
EHP = NW * PTP
CH = 80
NCH = PTP // CH
NBUF = 4
NG = NCH // NBUF
SNBUF = 2
SNG = NCH // SNBUF
N_PAD = 10240
ROWS_PER_SUB = N_PAD // NS

BN = 2000
BE = 2048

_MESH = plsc.VectorSubcoreMesh(core_axis_name="c", subcore_axis_name="s")


@functools.partial(
    pl.kernel,
    out_type=jax.ShapeDtypeStruct((EHP, H), jnp.float32),
    mesh=_MESH,
    scratch_types=(
        [pltpu.VMEM((PTP,), jnp.int32)] * 2
        + [pltpu.VMEM((CH, H), jnp.float32)] * (2 * NBUF)
        + [pltpu.SemaphoreType.DMA] * (3 * NBUF)
    ),
)
def _sc_gather(idx_hbm, a_hbm, b_hbm, out_hbm, row_v, col_v, *scr):
    abuf = scr[0:NBUF]
    bbuf = scr[NBUF:2 * NBUF]
    sga = scr[2 * NBUF:3 * NBUF]
    sgb = scr[3 * NBUF:4 * NBUF]
    ssa = scr[4 * NBUF:5 * NBUF]
    wid = lax.axis_index("s") * NC + lax.axis_index("c")
    pltpu.sync_copy(idx_hbm.at[0, wid], row_v)
    pltpu.sync_copy(idx_hbm.at[1, wid], col_v)
    base = wid * PTP

    def body(g, carry):
        c0 = g * NBUF
        for p in range(NBUF):
            @pl.when(g > 0)
            def _drain():
                pltpu.make_async_copy(abuf[p], out_hbm.at[pl.ds(base, CH)],
                                      ssa[p]).wait()
            pltpu.async_copy(a_hbm.at[row_v.at[pl.ds((c0 + p) * CH, CH)]],
                             abuf[p], sga[p])
            pltpu.async_copy(b_hbm.at[col_v.at[pl.ds((c0 + p) * CH, CH)]],
                             bbuf[p], sgb[p])
        for p in range(NBUF):
            off = base + (c0 + p) * CH
            pltpu.make_async_copy(a_hbm.at[row_v.at[pl.ds(0, CH)]], abuf[p],
                                  sga[p]).wait()
            pltpu.make_async_copy(b_hbm.at[col_v.at[pl.ds(0, CH)]], bbuf[p],
                                  sgb[p]).wait()

            def add_row(r, carry2):
                for j in range(H // 16):
                    abuf[p][r, pl.ds(j * 16, 16)] = (
                        abuf[p][r, pl.ds(j * 16, 16)]
                        + bbuf[p][r, pl.ds(j * 16, 16)])
                return carry2

            lax.fori_loop(0, CH, add_row, 0)
            pltpu.async_copy(abuf[p], out_hbm.at[pl.ds(off, CH)], ssa[p])
        return carry

    lax.fori_loop(0, NG, body, 0)
    for p in range(NBUF):
        pltpu.make_async_copy(abuf[p], out_hbm.at[pl.ds(base, CH)],
                              ssa[p]).wait()


@functools.partial(
    pl.kernel,
    out_type=jax.ShapeDtypeStruct((NC, N_PAD, H), jnp.float32),
    mesh=_MESH,
    scratch_types=(
        [pltpu.VMEM((NCH, CH), jnp.int32)]
        + [pltpu.VMEM((CH, H), jnp.float32)] * SNBUF
        + [pltpu.SemaphoreType.DMA] * (2 * SNBUF)
        + [pltpu.VMEM_SHARED((N_PAD, H), jnp.float32)]
    ),
)
def _sc_scatter(row_hbm, m2_hbm, init_hbm, out_hbm, row_v, *scr):
    mbuf = scr[0:SNBUF]
    sld = scr[SNBUF:2 * SNBUF]
    ssc = scr[2 * SNBUF:3 * SNBUF]
    agg_sp = scr[3 * SNBUF]
    cid = lax.axis_index("c")
    sid = lax.axis_index("s")
    wid = sid * NC + cid
    pltpu.sync_copy(row_hbm.at[wid], row_v)
    rs = sid * ROWS_PER_SUB
    pltpu.sync_copy(init_hbm.at[cid, pl.ds(rs, ROWS_PER_SUB)],
                    agg_sp.at[pl.ds(rs, ROWS_PER_SUB)])
    plsc.subcore_barrier()
    base = wid * PTP

    for p in range(SNBUF):
        pltpu.async_copy(m2_hbm.at[pl.ds(base + p * CH, CH)], mbuf[p],
                         sld[p])

    def body(g, carry):
        c0 = g * SNBUF
        for p in range(SNBUF):
            pltpu.make_async_copy(m2_hbm.at[pl.ds(base, CH)], mbuf[p],
                                  sld[p]).wait()
            pltpu.async_copy(mbuf[p], agg_sp.at[row_v.at[c0 + p]], ssc[p],
                             add=True)

            @pl.when(c0 + p + SNBUF < NCH)
            def _next():
                pltpu.make_async_copy(mbuf[p], agg_sp.at[row_v.at[c0 + p]],
                                      ssc[p]).wait()
                pltpu.async_copy(
                    m2_hbm.at[pl.ds(base + (c0 + SNBUF + p) * CH, CH)],
                    mbuf[p], sld[p])
        return carry

    lax.fori_loop(0, SNG, body, 0)
    for p in range(SNBUF):
        pltpu.make_async_copy(mbuf[p], agg_sp.at[row_v.at[0]],
                              ssc[p]).wait()
    plsc.subcore_barrier()
    pltpu.sync_copy(agg_sp.at[pl.ds(rs, ROWS_PER_SUB)],
                    out_hbm.at[cid, pl.ds(rs, ROWS_PER_SUB)])


def _full(shape):
    return pl.BlockSpec(shape, lambda n: (0,) * len(shape))


def _rows(bs, width):
    return pl.BlockSpec((bs, width), lambda n: (n, 0))


def _silu(v):
    return v * jax.nn.sigmoid(v)


def _embed_body(h_ref, we_ref, be_ref, ws_ref, wd_ref, e1_ref,
                x_ref, a_ref, b_ref):
    x = jnp.dot(h_ref[...], we_ref[...], preferred_element_type=jnp.float32)
    x = x + be_ref[...]
    x_ref[...] = x
    a_ref[...] = jnp.dot(x, ws_ref[...],
                         preferred_element_type=jnp.float32) + e1_ref[...]
    b_ref[...] = jnp.dot(x, wd_ref[...], preferred_element_type=jnp.float32)


def _tc_embed(h, emb_w, emb_b, wsrc, wdst, eb1_i):
    return pl.pallas_call(
        _embed_body,
        grid=(N // BN,),
        in_specs=[_rows(BN, H), _full((H, H)), _full((1, H)),
                  _full((H, H)), _full((H, H)), _full((1, H))],
        out_specs=[_rows(BN, H)] * 3,
        out_shape=[jax.ShapeDtypeStruct((N, H), jnp.float32)] * 3,
    )(h, emb_w, emb_b.reshape(1, H), wsrc, wdst, eb1_i.reshape(1, H))


def _edge_body(g_ref, ea_ref, we_ref, e2_ref, w2_ref, m2_ref):
    m1 = g_ref[...] + jnp.dot(
        ea_ref[...], we_ref[...], preferred_element_type=jnp.float32)
    m1 = _silu(m1).astype(jnp.bfloat16)
    m2 = jnp.dot(m1, w2_ref[...],
                 preferred_element_type=jnp.float32) + e2_ref[...]
    m2_ref[...] = _silu(m2)


def _tc_edge(gsum, edge_attr, we, eb2_i, ew2_i):
    return pl.pallas_call(
        _edge_body,
        grid=(EHP // BE,),
        in_specs=[
            _rows(BE, H),
            _rows(BE, DE), _full((DE, H)), _full((1, H)), _full((H, H)),
        ],
        out_specs=_rows(BE, H),
        out_shape=jax.ShapeDtypeStruct((EHP, H), jnp.float32),
    )(gsum, edge_attr, we, eb2_i.reshape(1, H),
      ew2_i.astype(jnp.bfloat16))


def _node_body(x_ref, agg_ref, w1x_ref, w1a_ref, n1_ref, w2_ref, n2_ref,
               ws_ref, wd_ref, e1_ref, x_out, a_out, b_out):
    agg = agg_ref[0] + agg_ref[1]
    t = (jnp.dot(x_ref[...], w1x_ref[...], preferred_element_type=jnp.float32)
         + jnp.dot(agg, w1a_ref[...], preferred_element_type=jnp.float32)
         + n1_ref[...])
    t = _silu(t)
    xn = x_ref[...] + jnp.dot(
        t, w2_ref[...], preferred_element_type=jnp.float32) + n2_ref[...]
    x_out[...] = xn
    a_out[...] = jnp.dot(xn, ws_ref[...],
                         preferred_element_type=jnp.float32) + e1_ref[...]
    b_out[...] = jnp.dot(xn, wd_ref[...], preferred_element_type=jnp.float32)


def _tc_node(x, agg2, nw1x, nw1a, nb1_i, nw2_i, nb2_i, wsrc, wdst, eb1_n):
    return pl.pallas_call(
        _node_body,
        grid=(N // BN,),
        in_specs=[
            _rows(BN, H),
            pl.BlockSpec((NC, BN, H), lambda n: (0, n, 0)),
            _full((H, H)), _full((H, H)), _full((1, H)),
            _full((H, H)), _full((1, H)),
            _full((H, H)), _full((H, H)), _full((1, H)),
        ],
        out_specs=[_rows(BN, H)] * 3,
        out_shape=[jax.ShapeDtypeStruct((N, H), jnp.float32)] * 3,
    )(x, agg2, nw1x, nw1a, nb1_i.reshape(1, H), nw2_i, nb2_i.reshape(1, H),
      wsrc, wdst, eb1_n.reshape(1, H))


def _node_final_body(x_ref, agg_ref, w1x_ref, w1a_ref, n1_ref, w2_ref,
                     n2_ref, wo_ref, bo_ref, o_ref):
    agg = agg_ref[0] + agg_ref[1]
    t = (jnp.dot(x_ref[...], w1x_ref[...], preferred_element_type=jnp.float32)
         + jnp.dot(agg, w1a_ref[...], preferred_element_type=jnp.float32)
         + n1_ref[...])
    t = _silu(t)
    xn = x_ref[...] + jnp.dot(
        t, w2_ref[...], preferred_element_type=jnp.float32) + n2_ref[...]
    o_ref[...] = jnp.dot(xn, wo_ref[...],
                         preferred_element_type=jnp.float32) + bo_ref[...]


def _tc_node_final(x, agg2, nw1x, nw1a, nb1_i, nw2_i, nb2_i, out_w, out_b):
    return pl.pallas_call(
        _node_final_body,
        grid=(N // BN,),
        in_specs=[
            _rows(BN, H),
            pl.BlockSpec((NC, BN, H), lambda n: (0, n, 0)),
            _full((H, H)), _full((H, H)), _full((1, H)),
            _full((H, H)), _full((1, H)),
            _full((H, H)), _full((1, H)),
        ],
        out_specs=_rows(BN, H),
        out_shape=jax.ShapeDtypeStruct((N, H), jnp.float32),
    )(x, agg2, nw1x, nw1a, nb1_i.reshape(1, H), nw2_i, nb2_i.reshape(1, H),
      out_w, out_b.reshape(1, H))


def kernel(h, edges, edge_attr, emb_w, emb_b, out_w, out_b,
           ew1, eb1, ew2, eb2, nw1, nb1, nw2, nb2):
    idx_halves = edges.reshape(2, 2, NW, PT)
    pads = ((0, 0), (0, 0), (0, PTP - PT))
    idx1 = jnp.pad(idx_halves[:, 0], pads)
    idx2 = jnp.pad(idx_halves[:, 1], pads)
    rpads = ((0, 0), (0, PTP - PT))
    row1 = jnp.pad(idx_halves[0, 0], rpads,
                   constant_values=N_PAD - 1).reshape(NW, NCH, CH)
    row2 = jnp.pad(idx_halves[0, 1], rpads,
                   constant_values=N_PAD - 1).reshape(NW, NCH, CH)
    eap = ((0, 0), (0, PTP - PT), (0, 0))
    ea_halves = edge_attr.reshape(2, NW, PT, DE)
    ea1 = jnp.pad(ea_halves[0], eap).reshape(EHP, DE)
    ea2 = jnp.pad(ea_halves[1], eap).reshape(EHP, DE)
    zeros2 = jnp.zeros((NC, N_PAD, H), jnp.float32)

    x, a, b = _tc_embed(h, emb_w, emb_b, ew1[0, :H], ew1[0, H:2 * H], eb1[0])
    for i in range(L):
        g1 = _sc_gather(idx1, a, b)
        g2 = _sc_gather(idx2, a, b)
        m21 = _tc_edge(g1, ea1, ew1[i, 2 * H:], eb2[i], ew2[i])
        m22 = _tc_edge(g2, ea2, ew1[i, 2 * H:], eb2[i], ew2[i])
        s1 = _sc_scatter(row1, m21, zeros2)
        agg2 = _sc_scatter(row2, m22, s1)
        if i + 1 < L:
            x, a, b = _tc_node(x, agg2, nw1[i, :H], nw1[i, H:], nb1[i],
                               nw2[i], nb2[i], ew1[i + 1, :H],
                               ew1[i + 1, H:2 * H], eb1[i + 1])
        else:
            out = _tc_node_final(x, agg2, nw1[i, :H], nw1[i, H:], nb1[i],
                                 nw2[i], nb2[i], out_w, out_b)
    return out


# speedup vs baseline: 1.8487x; 1.8487x over previous
import functools

import jax
import jax.numpy as jnp
from jax import lax
from jax.experimental import pallas as pl
from jax.experimental.pallas import tpu as pltpu
from jax.experimental.pallas import tpu_sc as plsc

N = 10000
E = 320000
H = 128
DE = 4
L = 4

NC = 2
NS = 16
NW = NC * NS
EH = E // 2
PT = EH // NW
CH = 40
NCH = PT // CH
NBUF = 5
NG = NCH // NBUF
SNBUF = 2
SNG = (NCH - 1) // SNBUF
N_PAD = 10240
ROWS_PER_SUB = N_PAD // NS

BN = 2000
BE = 4000

_MESH = plsc.VectorSubcoreMesh(core_axis_name="c", subcore_axis_name="s")


@functools.partial(
    pl.kernel,
    out_type=jax.ShapeDtypeStruct((EH, H), jnp.float32),
    mesh=_MESH,
    scratch_types=(
        [pltpu.VMEM((PT,), jnp.int32)] * 2
        + [pltpu.VMEM((CH, H), jnp.float32)] * (2 * NBUF)
        + [pltpu.SemaphoreType.DMA] * (3 * NBUF)
    ),
)
def _sc_gather(idx_hbm, a_hbm, b_hbm, out_hbm, row_v, col_v, *scr):
    abuf = scr[0:NBUF]
    bbuf = scr[NBUF:2 * NBUF]
    sga = scr[2 * NBUF:3 * NBUF]
    sgb = scr[3 * NBUF:4 * NBUF]
    ssa = scr[4 * NBUF:5 * NBUF]
    wid = lax.axis_index("s") * NC + lax.axis_index("c")
    pltpu.sync_copy(idx_hbm.at[0, wid], row_v)
    pltpu.sync_copy(idx_hbm.at[1, wid], col_v)
    base = wid * PT

    def body(g, carry):
        c0 = g * NBUF
        for p in range(NBUF):
            @pl.when(g > 0)
            def _drain():
                pltpu.make_async_copy(abuf[p], out_hbm.at[pl.ds(base, CH)],
                                      ssa[p]).wait()
            pltpu.async_copy(a_hbm.at[row_v.at[pl.ds((c0 + p) * CH, CH)]],
                             abuf[p], sga[p])
            pltpu.async_copy(b_hbm.at[col_v.at[pl.ds((c0 + p) * CH, CH)]],
                             bbuf[p], sgb[p])
        for p in range(NBUF):
            off = base + (c0 + p) * CH
            pltpu.make_async_copy(a_hbm.at[row_v.at[pl.ds(0, CH)]], abuf[p],
                                  sga[p]).wait()
            pltpu.make_async_copy(b_hbm.at[col_v.at[pl.ds(0, CH)]], bbuf[p],
                                  sgb[p]).wait()

            def add_row(r, carry2):
                for j in range(H // 16):
                    abuf[p][r, pl.ds(j * 16, 16)] = (
                        abuf[p][r, pl.ds(j * 16, 16)]
                        + bbuf[p][r, pl.ds(j * 16, 16)])
                return carry2

            lax.fori_loop(0, CH, add_row, 0)
            pltpu.async_copy(abuf[p], out_hbm.at[pl.ds(off, CH)], ssa[p])
        return carry

    lax.fori_loop(0, NG, body, 0)
    for p in range(NBUF):
        pltpu.make_async_copy(abuf[p], out_hbm.at[pl.ds(base, CH)],
                              ssa[p]).wait()


@functools.partial(
    pl.kernel,
    out_type=jax.ShapeDtypeStruct((NC, N_PAD, H), jnp.float32),
    mesh=_MESH,
    scratch_types=(
        [pltpu.VMEM((NCH, CH), jnp.int32)]
        + [pltpu.VMEM((CH, H), jnp.float32)] * SNBUF
        + [pltpu.SemaphoreType.DMA] * SNBUF
        + [pltpu.VMEM_SHARED((N_PAD, H), jnp.float32)]
    ),
)
def _sc_scatter(row_hbm, m2_hbm, init_hbm, out_hbm, row_v, *scr):
    mbuf = scr[0:SNBUF]
    sld = scr[SNBUF:2 * SNBUF]
    agg_sp = scr[2 * SNBUF]
    cid = lax.axis_index("c")
    sid = lax.axis_index("s")
    wid = sid * NC + cid
    pltpu.sync_copy(row_hbm.at[wid], row_v)
    rs = sid * ROWS_PER_SUB
    pltpu.sync_copy(init_hbm.at[cid, pl.ds(rs, ROWS_PER_SUB)],
                    agg_sp.at[pl.ds(rs, ROWS_PER_SUB)])
    plsc.subcore_barrier()
    base = wid * PT

    for p in range(SNBUF):
        pltpu.async_copy(m2_hbm.at[pl.ds(base + p * CH, CH)], mbuf[p],
                         sld[p])

    def body(g, carry):
        c0 = g * SNBUF
        for p in range(SNBUF):
            pltpu.make_async_copy(m2_hbm.at[pl.ds(base, CH)], mbuf[p],
                                  sld[p]).wait()
            pltpu.sync_copy(mbuf[p], agg_sp.at[row_v.at[c0 + p]], add=True)

            @pl.when(c0 + p + SNBUF < NCH)
            def _next():
                pltpu.async_copy(
                    m2_hbm.at[pl.ds(base + (c0 + SNBUF + p) * CH, CH)],
                    mbuf[p], sld[p])
        return carry

    lax.fori_loop(0, SNG, body, 0)
    pltpu.make_async_copy(m2_hbm.at[pl.ds(base, CH)], mbuf[0], sld[0]).wait()
    pltpu.sync_copy(mbuf[0], agg_sp.at[row_v.at[NCH - 1]], add=True)
    plsc.subcore_barrier()
    pltpu.sync_copy(agg_sp.at[pl.ds(rs, ROWS_PER_SUB)],
                    out_hbm.at[cid, pl.ds(rs, ROWS_PER_SUB)])


def _full(shape):
    return pl.BlockSpec(shape, lambda n: (0,) * len(shape))


def _rows(bs, width):
    return pl.BlockSpec((bs, width), lambda n: (n, 0))


def _silu(v):
    return v * jax.nn.sigmoid(v)


def _embed_body(h_ref, we_ref, be_ref, ws_ref, wd_ref, e1_ref,
                x_ref, a_ref, b_ref):
    x = jnp.dot(h_ref[...], we_ref[...], preferred_element_type=jnp.float32)
    x = x + be_ref[...]
    x_ref[...] = x
    a_ref[...] = jnp.dot(x, ws_ref[...],
                         preferred_element_type=jnp.float32) + e1_ref[...]
    b_ref[...] = jnp.dot(x, wd_ref[...], preferred_element_type=jnp.float32)


def _tc_embed(h, emb_w, emb_b, wsrc, wdst, eb1_i):
    return pl.pallas_call(
        _embed_body,
        grid=(N // BN,),
        in_specs=[_rows(BN, H), _full((H, H)), _full((1, H)),
                  _full((H, H)), _full((H, H)), _full((1, H))],
        out_specs=[_rows(BN, H)] * 3,
        out_shape=[jax.ShapeDtypeStruct((N, H), jnp.float32)] * 3,
    )(h, emb_w, emb_b.reshape(1, H), wsrc, wdst, eb1_i.reshape(1, H))


def _edge_body(g_ref, ea_ref, we_ref, e2_ref, w2_ref, m2_ref):
    m1 = g_ref[...] + jnp.dot(
        ea_ref[...], we_ref[...], preferred_element_type=jnp.float32)
    m1 = _silu(m1).astype(jnp.bfloat16)
    m2 = jnp.dot(m1, w2_ref[...],
                 preferred_element_type=jnp.float32) + e2_ref[...]
    m2_ref[...] = _silu(m2)


def _tc_edge(gsum, edge_attr, we, eb2_i, ew2_i):
    return pl.pallas_call(
        _edge_body,
        grid=(EH // BE,),
        in_specs=[
            _rows(BE, H),
            _rows(BE, DE), _full((DE, H)), _full((1, H)), _full((H, H)),
        ],
        out_specs=_rows(BE, H),
        out_shape=jax.ShapeDtypeStruct((EH, H), jnp.float32),
    )(gsum, edge_attr, we, eb2_i.reshape(1, H),
      ew2_i.astype(jnp.bfloat16))


def _node_body(x_ref, agg_ref, w1x_ref, w1a_ref, n1_ref, w2_ref, n2_ref,
               ws_ref, wd_ref, e1_ref, x_out, a_out, b_out):
    agg = agg_ref[0] + agg_ref[1]
    t = (jnp.dot(x_ref[...], w1x_ref[...], preferred_element_type=jnp.float32)
         + jnp.dot(agg, w1a_ref[...], preferred_element_type=jnp.float32)
         + n1_ref[...])
    t = _silu(t)
    xn = x_ref[...] + jnp.dot(
        t, w2_ref[...], preferred_element_type=jnp.float32) + n2_ref[...]
    x_out[...] = xn
    a_out[...] = jnp.dot(xn, ws_ref[...],
                         preferred_element_type=jnp.float32) + e1_ref[...]
    b_out[...] = jnp.dot(xn, wd_ref[...], preferred_element_type=jnp.float32)


def _tc_node(x, agg2, nw1x, nw1a, nb1_i, nw2_i, nb2_i, wsrc, wdst, eb1_n):
    return pl.pallas_call(
        _node_body,
        grid=(N // BN,),
        in_specs=[
            _rows(BN, H),
            pl.BlockSpec((NC, BN, H), lambda n: (0, n, 0)),
            _full((H, H)), _full((H, H)), _full((1, H)),
            _full((H, H)), _full((1, H)),
            _full((H, H)), _full((H, H)), _full((1, H)),
        ],
        out_specs=[_rows(BN, H)] * 3,
        out_shape=[jax.ShapeDtypeStruct((N, H), jnp.float32)] * 3,
    )(x, agg2, nw1x, nw1a, nb1_i.reshape(1, H), nw2_i, nb2_i.reshape(1, H),
      wsrc, wdst, eb1_n.reshape(1, H))


def _node_final_body(x_ref, agg_ref, w1x_ref, w1a_ref, n1_ref, w2_ref,
                     n2_ref, wo_ref, bo_ref, o_ref):
    agg = agg_ref[0] + agg_ref[1]
    t = (jnp.dot(x_ref[...], w1x_ref[...], preferred_element_type=jnp.float32)
         + jnp.dot(agg, w1a_ref[...], preferred_element_type=jnp.float32)
         + n1_ref[...])
    t = _silu(t)
    xn = x_ref[...] + jnp.dot(
        t, w2_ref[...], preferred_element_type=jnp.float32) + n2_ref[...]
    o_ref[...] = jnp.dot(xn, wo_ref[...],
                         preferred_element_type=jnp.float32) + bo_ref[...]


def _tc_node_final(x, agg2, nw1x, nw1a, nb1_i, nw2_i, nb2_i, out_w, out_b):
    return pl.pallas_call(
        _node_final_body,
        grid=(N // BN,),
        in_specs=[
            _rows(BN, H),
            pl.BlockSpec((NC, BN, H), lambda n: (0, n, 0)),
            _full((H, H)), _full((H, H)), _full((1, H)),
            _full((H, H)), _full((1, H)),
            _full((H, H)), _full((1, H)),
        ],
        out_specs=_rows(BN, H),
        out_shape=jax.ShapeDtypeStruct((N, H), jnp.float32),
    )(x, agg2, nw1x, nw1a, nb1_i.reshape(1, H), nw2_i, nb2_i.reshape(1, H),
      out_w, out_b.reshape(1, H))


def kernel(h, edges, edge_attr, emb_w, emb_b, out_w, out_b,
           ew1, eb1, ew2, eb2, nw1, nb1, nw2, nb2):
    idx_halves = edges.reshape(2, 2, EH)
    idx1 = idx_halves[:, 0].reshape(2, NW, PT)
    idx2 = idx_halves[:, 1].reshape(2, NW, PT)
    row1 = idx1[0].reshape(NW, NCH, CH)
    row2 = idx2[0].reshape(NW, NCH, CH)
    ea1 = edge_attr[:EH]
    ea2 = edge_attr[EH:]
    zeros2 = jnp.zeros((NC, N_PAD, H), jnp.float32)

    x, a, b = _tc_embed(h, emb_w, emb_b, ew1[0, :H], ew1[0, H:2 * H], eb1[0])
    for i in range(L):
        g1 = _sc_gather(idx1, a, b)
        g2 = _sc_gather(idx2, a, b)
        m21 = _tc_edge(g1, ea1, ew1[i, 2 * H:], eb2[i], ew2[i])
        m22 = _tc_edge(g2, ea2, ew1[i, 2 * H:], eb2[i], ew2[i])
        s1 = _sc_scatter(row1, m21, zeros2)
        agg2 = _sc_scatter(row2, m22, s1)
        if i + 1 < L:
            x, a, b = _tc_node(x, agg2, nw1[i, :H], nw1[i, H:], nb1[i],
                               nw2[i], nb2[i], ew1[i + 1, :H],
                               ew1[i + 1, H:2 * H], eb1[i + 1])
        else:
            out = _tc_node_final(x, agg2, nw1[i, :H], nw1[i, H:], nb1[i],
                                 nw2[i], nb2[i], out_w, out_b)
    return out


# speedup vs baseline: 2.0002x; 1.0819x over previous
import functools

import jax
import jax.numpy as jnp
from jax import lax
from jax.experimental import pallas as pl
from jax.experimental.pallas import tpu as pltpu
from jax.experimental.pallas import tpu_sc as plsc

N = 10000
E = 320000
H = 128
DE = 4
L = 4

NC = 2
NS = 16
NW = NC * NS
EH = E // 2
PT = EH // NW
CH = 40
NCH = PT // CH
NBUF = 5
NG = NCH // NBUF
SNBUF = 4
SNG = (NCH - 1) // SNBUF
N_PAD = 10240
ROWS_PER_SUB = N_PAD // NS

BN = 2000
BE = 4000

_MESH = plsc.VectorSubcoreMesh(core_axis_name="c", subcore_axis_name="s")


@functools.partial(
    pl.kernel,
    out_type=jax.ShapeDtypeStruct((EH, H), jnp.float32),
    mesh=_MESH,
    scratch_types=(
        [pltpu.VMEM((PT,), jnp.int32)] * 2
        + [pltpu.VMEM((CH, H), jnp.float32)] * (2 * NBUF)
        + [pltpu.SemaphoreType.DMA] * (3 * NBUF)
    ),
)
def _sc_gather(idx_hbm, a_hbm, b_hbm, out_hbm, row_v, col_v, *scr):
    abuf = scr[0:NBUF]
    bbuf = scr[NBUF:2 * NBUF]
    sga = scr[2 * NBUF:3 * NBUF]
    sgb = scr[3 * NBUF:4 * NBUF]
    ssa = scr[4 * NBUF:5 * NBUF]
    wid = lax.axis_index("s") * NC + lax.axis_index("c")
    pltpu.sync_copy(idx_hbm.at[0, wid], row_v)
    pltpu.sync_copy(idx_hbm.at[1, wid], col_v)
    base = wid * PT

    def body(g, carry):
        c0 = g * NBUF
        for p in range(NBUF):
            @pl.when(g > 0)
            def _drain():
                pltpu.make_async_copy(abuf[p], out_hbm.at[pl.ds(base, CH)],
                                      ssa[p]).wait()
            pltpu.async_copy(a_hbm.at[row_v.at[pl.ds((c0 + p) * CH, CH)]],
                             abuf[p], sga[p])
            pltpu.async_copy(b_hbm.at[col_v.at[pl.ds((c0 + p) * CH, CH)]],
                             bbuf[p], sgb[p])
        for p in range(NBUF):
            off = base + (c0 + p) * CH
            pltpu.make_async_copy(a_hbm.at[row_v.at[pl.ds(0, CH)]], abuf[p],
                                  sga[p]).wait()
            pltpu.make_async_copy(b_hbm.at[col_v.at[pl.ds(0, CH)]], bbuf[p],
                                  sgb[p]).wait()

            def add_row(r, carry2):
                for j in range(H // 16):
                    abuf[p][r, pl.ds(j * 16, 16)] = (
                        abuf[p][r, pl.ds(j * 16, 16)]
                        + bbuf[p][r, pl.ds(j * 16, 16)])
                return carry2

            lax.fori_loop(0, CH, add_row, 0)
            pltpu.async_copy(abuf[p], out_hbm.at[pl.ds(off, CH)], ssa[p])
        return carry

    lax.fori_loop(0, NG, body, 0)
    for p in range(NBUF):
        pltpu.make_async_copy(abuf[p], out_hbm.at[pl.ds(base, CH)],
                              ssa[p]).wait()


@functools.partial(
    pl.kernel,
    out_type=jax.ShapeDtypeStruct((NC, N_PAD, H), jnp.float32),
    mesh=_MESH,
    scratch_types=(
        [pltpu.VMEM((NCH, CH), jnp.int32)]
        + [pltpu.VMEM((CH, H), jnp.float32)] * SNBUF
        + [pltpu.SemaphoreType.DMA] * SNBUF
        + [pltpu.VMEM_SHARED((N_PAD, H), jnp.float32)]
    ),
)
def _sc_scatter(row_hbm, m2_hbm, init_hbm, out_hbm, row_v, *scr):
    mbuf = scr[0:SNBUF]
    sld = scr[SNBUF:2 * SNBUF]
    agg_sp = scr[2 * SNBUF]
    cid = lax.axis_index("c")
    sid = lax.axis_index("s")
    wid = sid * NC + cid
    pltpu.sync_copy(row_hbm.at[wid], row_v)
    rs = sid * ROWS_PER_SUB
    pltpu.sync_copy(init_hbm.at[cid, pl.ds(rs, ROWS_PER_SUB)],
                    agg_sp.at[pl.ds(rs, ROWS_PER_SUB)])
    plsc.subcore_barrier()
    base = wid * PT

    for p in range(SNBUF):
        pltpu.async_copy(m2_hbm.at[pl.ds(base + p * CH, CH)], mbuf[p],
                         sld[p])

    def body(g, carry):
        c0 = g * SNBUF
        for p in range(SNBUF):
            pltpu.make_async_copy(m2_hbm.at[pl.ds(base, CH)], mbuf[p],
                                  sld[p]).wait()
            pltpu.sync_copy(mbuf[p], agg_sp.at[row_v.at[c0 + p]], add=True)

            @pl.when(c0 + p + SNBUF < NCH)
            def _next():
                pltpu.async_copy(
                    m2_hbm.at[pl.ds(base + (c0 + SNBUF + p) * CH, CH)],
                    mbuf[p], sld[p])
        return carry

    lax.fori_loop(0, SNG, body, 0)
    pltpu.make_async_copy(m2_hbm.at[pl.ds(base, CH)], mbuf[0], sld[0]).wait()
    pltpu.sync_copy(mbuf[0], agg_sp.at[row_v.at[NCH - 1]], add=True)
    plsc.subcore_barrier()
    pltpu.sync_copy(agg_sp.at[pl.ds(rs, ROWS_PER_SUB)],
                    out_hbm.at[cid, pl.ds(rs, ROWS_PER_SUB)])


def _full(shape):
    return pl.BlockSpec(shape, lambda n: (0,) * len(shape))


def _rows(bs, width):
    return pl.BlockSpec((bs, width), lambda n: (n, 0))


def _silu(v):
    return v * jax.nn.sigmoid(v)


def _embed_body(h_ref, we_ref, be_ref, ws_ref, wd_ref, e1_ref,
                x_ref, a_ref, b_ref):
    x = jnp.dot(h_ref[...], we_ref[...], preferred_element_type=jnp.float32)
    x = x + be_ref[...]
    x_ref[...] = x
    a_ref[...] = jnp.dot(x, ws_ref[...],
                         preferred_element_type=jnp.float32) + e1_ref[...]
    b_ref[...] = jnp.dot(x, wd_ref[...], preferred_element_type=jnp.float32)


def _tc_embed(h, emb_w, emb_b, wsrc, wdst, eb1_i):
    return pl.pallas_call(
        _embed_body,
        grid=(N // BN,),
        in_specs=[_rows(BN, H), _full((H, H)), _full((1, H)),
                  _full((H, H)), _full((H, H)), _full((1, H))],
        out_specs=[_rows(BN, H)] * 3,
        out_shape=[jax.ShapeDtypeStruct((N, H), jnp.float32)] * 3,
    )(h, emb_w, emb_b.reshape(1, H), wsrc, wdst, eb1_i.reshape(1, H))


def _edge_body(g_ref, ea_ref, we_ref, e2_ref, w2_ref, m2_ref):
    m1 = g_ref[...] + jnp.dot(
        ea_ref[...], we_ref[...], preferred_element_type=jnp.float32)
    m1 = _silu(m1).astype(jnp.bfloat16)
    m2 = jnp.dot(m1, w2_ref[...],
                 preferred_element_type=jnp.float32) + e2_ref[...]
    m2_ref[...] = _silu(m2)


def _tc_edge(gsum, edge_attr, we, eb2_i, ew2_i):
    return pl.pallas_call(
        _edge_body,
        grid=(EH // BE,),
        in_specs=[
            _rows(BE, H),
            _rows(BE, DE), _full((DE, H)), _full((1, H)), _full((H, H)),
        ],
        out_specs=_rows(BE, H),
        out_shape=jax.ShapeDtypeStruct((EH, H), jnp.float32),
    )(gsum, edge_attr, we, eb2_i.reshape(1, H),
      ew2_i.astype(jnp.bfloat16))


def _node_body(x_ref, agg_ref, w1x_ref, w1a_ref, n1_ref, w2_ref, n2_ref,
               ws_ref, wd_ref, e1_ref, x_out, a_out, b_out):
    agg = agg_ref[0] + agg_ref[1]
    t = (jnp.dot(x_ref[...], w1x_ref[...], preferred_element_type=jnp.float32)
         + jnp.dot(agg, w1a_ref[...], preferred_element_type=jnp.float32)
         + n1_ref[...])
    t = _silu(t)
    xn = x_ref[...] + jnp.dot(
        t, w2_ref[...], preferred_element_type=jnp.float32) + n2_ref[...]
    x_out[...] = xn
    a_out[...] = jnp.dot(xn, ws_ref[...],
                         preferred_element_type=jnp.float32) + e1_ref[...]
    b_out[...] = jnp.dot(xn, wd_ref[...], preferred_element_type=jnp.float32)


def _tc_node(x, agg2, nw1x, nw1a, nb1_i, nw2_i, nb2_i, wsrc, wdst, eb1_n):
    return pl.pallas_call(
        _node_body,
        grid=(N // BN,),
        in_specs=[
            _rows(BN, H),
            pl.BlockSpec((NC, BN, H), lambda n: (0, n, 0)),
            _full((H, H)), _full((H, H)), _full((1, H)),
            _full((H, H)), _full((1, H)),
            _full((H, H)), _full((H, H)), _full((1, H)),
        ],
        out_specs=[_rows(BN, H)] * 3,
        out_shape=[jax.ShapeDtypeStruct((N, H), jnp.float32)] * 3,
    )(x, agg2, nw1x, nw1a, nb1_i.reshape(1, H), nw2_i, nb2_i.reshape(1, H),
      wsrc, wdst, eb1_n.reshape(1, H))


def _node_final_body(x_ref, agg_ref, w1x_ref, w1a_ref, n1_ref, w2_ref,
                     n2_ref, wo_ref, bo_ref, o_ref):
    agg = agg_ref[0] + agg_ref[1]
    t = (jnp.dot(x_ref[...], w1x_ref[...], preferred_element_type=jnp.float32)
         + jnp.dot(agg, w1a_ref[...], preferred_element_type=jnp.float32)
         + n1_ref[...])
    t = _silu(t)
    xn = x_ref[...] + jnp.dot(
        t, w2_ref[...], preferred_element_type=jnp.float32) + n2_ref[...]
    o_ref[...] = jnp.dot(xn, wo_ref[...],
                         preferred_element_type=jnp.float32) + bo_ref[...]


def _tc_node_final(x, agg2, nw1x, nw1a, nb1_i, nw2_i, nb2_i, out_w, out_b):
    return pl.pallas_call(
        _node_final_body,
        grid=(N // BN,),
        in_specs=[
            _rows(BN, H),
            pl.BlockSpec((NC, BN, H), lambda n: (0, n, 0)),
            _full((H, H)), _full((H, H)), _full((1, H)),
            _full((H, H)), _full((1, H)),
            _full((H, H)), _full((1, H)),
        ],
        out_specs=_rows(BN, H),
        out_shape=jax.ShapeDtypeStruct((N, H), jnp.float32),
    )(x, agg2, nw1x, nw1a, nb1_i.reshape(1, H), nw2_i, nb2_i.reshape(1, H),
      out_w, out_b.reshape(1, H))


def kernel(h, edges, edge_attr, emb_w, emb_b, out_w, out_b,
           ew1, eb1, ew2, eb2, nw1, nb1, nw2, nb2):
    idx_halves = edges.reshape(2, 2, EH)
    idx1 = idx_halves[:, 0].reshape(2, NW, PT)
    idx2 = idx_halves[:, 1].reshape(2, NW, PT)
    row1 = idx1[0].reshape(NW, NCH, CH)
    row2 = idx2[0].reshape(NW, NCH, CH)
    ea1 = edge_attr[:EH]
    ea2 = edge_attr[EH:]
    zeros2 = jnp.zeros((NC, N_PAD, H), jnp.float32)

    x, a, b = _tc_embed(h, emb_w, emb_b, ew1[0, :H], ew1[0, H:2 * H], eb1[0])
    for i in range(L):
        g1 = _sc_gather(idx1, a, b)
        g2 = _sc_gather(idx2, a, b)
        m21 = _tc_edge(g1, ea1, ew1[i, 2 * H:], eb2[i], ew2[i])
        m22 = _tc_edge(g2, ea2, ew1[i, 2 * H:], eb2[i], ew2[i])
        s1 = _sc_scatter(row1, m21, zeros2)
        agg2 = _sc_scatter(row2, m22, s1)
        if i + 1 < L:
            x, a, b = _tc_node(x, agg2, nw1[i, :H], nw1[i, H:], nb1[i],
                               nw2[i], nb2[i], ew1[i + 1, :H],
                               ew1[i + 1, H:2 * H], eb1[i + 1])
        else:
            out = _tc_node_final(x, agg2, nw1[i, :H], nw1[i, H:], nb1[i],
                                 nw2[i], nb2[i], out_w, out_b)
    return out


# speedup vs baseline: 2.0562x; 1.0280x over previous
import functools

import jax
import jax.numpy as jnp
from jax import lax
from jax.experimental import pallas as pl
from jax.experimental.pallas import tpu as pltpu
from jax.experimental.pallas import tpu_sc as plsc

N = 10000
E = 320000
H = 128
DE = 4
L = 4

NC = 2
NS = 16
NW = NC * NS
EH = E // 2
PT = EH // NW
CH = 40
NCH = PT // CH
NBUF = 5
NG = NCH // NBUF
SNBUF = 4
SNG = (NCH - 1) // SNBUF
N_PAD = 10240
ROWS_PER_SUB = N_PAD // NS

BN = 2000
BE = 4000

_MESH = plsc.VectorSubcoreMesh(core_axis_name="c", subcore_axis_name="s")


@functools.partial(
    pl.kernel,
    out_type=jax.ShapeDtypeStruct((EH, H), jnp.float32),
    mesh=_MESH,
    scratch_types=(
        [pltpu.VMEM((PT,), jnp.int32)] * 2
        + [pltpu.VMEM((CH, H), jnp.float32)] * (2 * NBUF)
        + [pltpu.SemaphoreType.DMA] * (3 * NBUF)
    ),
)
def _sc_gather(idx_hbm, a_hbm, b_hbm, out_hbm, row_v, col_v, *scr):
    abuf = scr[0:NBUF]
    bbuf = scr[NBUF:2 * NBUF]
    sga = scr[2 * NBUF:3 * NBUF]
    sgb = scr[3 * NBUF:4 * NBUF]
    ssa = scr[4 * NBUF:5 * NBUF]
    wid = lax.axis_index("s") * NC + lax.axis_index("c")
    pltpu.sync_copy(idx_hbm.at[0, wid], row_v)
    pltpu.sync_copy(idx_hbm.at[1, wid], col_v)
    base = wid * PT

    for p in range(NBUF):
        pltpu.async_copy(a_hbm.at[row_v.at[pl.ds(p * CH, CH)]],
                         abuf[p], sga[p])
        pltpu.async_copy(b_hbm.at[col_v.at[pl.ds(p * CH, CH)]],
                         bbuf[p], sgb[p])

    def body(g, carry):
        c0 = g * NBUF
        for p in range(NBUF):
            c = c0 + p
            pltpu.make_async_copy(a_hbm.at[row_v.at[pl.ds(0, CH)]], abuf[p],
                                  sga[p]).wait()
            pltpu.make_async_copy(b_hbm.at[col_v.at[pl.ds(0, CH)]], bbuf[p],
                                  sgb[p]).wait()

            def add_row(r, carry2):
                for j in range(H // 16):
                    abuf[p][r, pl.ds(j * 16, 16)] = (
                        abuf[p][r, pl.ds(j * 16, 16)]
                        + bbuf[p][r, pl.ds(j * 16, 16)])
                return carry2

            lax.fori_loop(0, CH, add_row, 0)
            pltpu.async_copy(abuf[p], out_hbm.at[pl.ds(base + c * CH, CH)],
                             ssa[p])

            @pl.when(c + NBUF < NCH)
            def _refill():
                pltpu.make_async_copy(abuf[p], out_hbm.at[pl.ds(base, CH)],
                                      ssa[p]).wait()
                pltpu.async_copy(
                    a_hbm.at[row_v.at[pl.ds((c + NBUF) * CH, CH)]],
                    abuf[p], sga[p])
                pltpu.async_copy(
                    b_hbm.at[col_v.at[pl.ds((c + NBUF) * CH, CH)]],
                    bbuf[p], sgb[p])
        return carry

    lax.fori_loop(0, NG, body, 0)
    for p in range(NBUF):
        pltpu.make_async_copy(abuf[p], out_hbm.at[pl.ds(base, CH)],
                              ssa[p]).wait()


@functools.partial(
    pl.kernel,
    out_type=jax.ShapeDtypeStruct((NC, N_PAD, H), jnp.float32),
    mesh=_MESH,
    scratch_types=(
        [pltpu.VMEM((NCH, CH), jnp.int32)]
        + [pltpu.VMEM((CH, H), jnp.float32)] * SNBUF
        + [pltpu.SemaphoreType.DMA] * SNBUF
        + [pltpu.VMEM_SHARED((N_PAD, H), jnp.float32)]
    ),
)
def _sc_scatter(row_hbm, m2_hbm, init_hbm, out_hbm, row_v, *scr):
    mbuf = scr[0:SNBUF]
    sld = scr[SNBUF:2 * SNBUF]
    agg_sp = scr[2 * SNBUF]
    cid = lax.axis_index("c")
    sid = lax.axis_index("s")
    wid = sid * NC + cid
    pltpu.sync_copy(row_hbm.at[wid], row_v)
    rs = sid * ROWS_PER_SUB
    pltpu.sync_copy(init_hbm.at[cid, pl.ds(rs, ROWS_PER_SUB)],
                    agg_sp.at[pl.ds(rs, ROWS_PER_SUB)])
    plsc.subcore_barrier()
    base = wid * PT

    for p in range(SNBUF):
        pltpu.async_copy(m2_hbm.at[pl.ds(base + p * CH, CH)], mbuf[p],
                         sld[p])

    def body(g, carry):
        c0 = g * SNBUF
        for p in range(SNBUF):
            pltpu.make_async_copy(m2_hbm.at[pl.ds(base, CH)], mbuf[p],
                                  sld[p]).wait()
            pltpu.sync_copy(mbuf[p], agg_sp.at[row_v.at[c0 + p]], add=True)

            @pl.when(c0 + p + SNBUF < NCH)
            def _next():
                pltpu.async_copy(
                    m2_hbm.at[pl.ds(base + (c0 + SNBUF + p) * CH, CH)],
                    mbuf[p], sld[p])
        return carry

    lax.fori_loop(0, SNG, body, 0)
    pltpu.make_async_copy(m2_hbm.at[pl.ds(base, CH)], mbuf[0], sld[0]).wait()
    pltpu.sync_copy(mbuf[0], agg_sp.at[row_v.at[NCH - 1]], add=True)
    plsc.subcore_barrier()
    pltpu.sync_copy(agg_sp.at[pl.ds(rs, ROWS_PER_SUB)],
                    out_hbm.at[cid, pl.ds(rs, ROWS_PER_SUB)])


def _full(shape):
    return pl.BlockSpec(shape, lambda n: (0,) * len(shape))


def _rows(bs, width):
    return pl.BlockSpec((bs, width), lambda n: (n, 0))


def _silu(v):
    return v * jax.nn.sigmoid(v)


def _embed_body(h_ref, we_ref, be_ref, ws_ref, wd_ref, e1_ref,
                x_ref, a_ref, b_ref):
    x = jnp.dot(h_ref[...], we_ref[...], preferred_element_type=jnp.float32)
    x = x + be_ref[...]
    x_ref[...] = x
    a_ref[...] = jnp.dot(x, ws_ref[...],
                         preferred_element_type=jnp.float32) + e1_ref[...]
    b_ref[...] = jnp.dot(x, wd_ref[...], preferred_element_type=jnp.float32)


def _tc_embed(h, emb_w, emb_b, wsrc, wdst, eb1_i):
    return pl.pallas_call(
        _embed_body,
        grid=(N // BN,),
        in_specs=[_rows(BN, H), _full((H, H)), _full((1, H)),
                  _full((H, H)), _full((H, H)), _full((1, H))],
        out_specs=[_rows(BN, H)] * 3,
        out_shape=[jax.ShapeDtypeStruct((N, H), jnp.float32)] * 3,
    )(h, emb_w, emb_b.reshape(1, H), wsrc, wdst, eb1_i.reshape(1, H))


def _edge_body(g_ref, ea_ref, we_ref, e2_ref, w2_ref, m2_ref):
    m1 = g_ref[...] + jnp.dot(
        ea_ref[...], we_ref[...], preferred_element_type=jnp.float32)
    m1 = _silu(m1).astype(jnp.bfloat16)
    m2 = jnp.dot(m1, w2_ref[...],
                 preferred_element_type=jnp.float32) + e2_ref[...]
    m2_ref[...] = _silu(m2)


def _tc_edge(gsum, edge_attr, we, eb2_i, ew2_i):
    return pl.pallas_call(
        _edge_body,
        grid=(EH // BE,),
        in_specs=[
            _rows(BE, H),
            _rows(BE, DE), _full((DE, H)), _full((1, H)), _full((H, H)),
        ],
        out_specs=_rows(BE, H),
        out_shape=jax.ShapeDtypeStruct((EH, H), jnp.float32),
    )(gsum, edge_attr, we, eb2_i.reshape(1, H),
      ew2_i.astype(jnp.bfloat16))


def _node_body(x_ref, agg_ref, w1x_ref, w1a_ref, n1_ref, w2_ref, n2_ref,
               ws_ref, wd_ref, e1_ref, x_out, a_out, b_out):
    agg = agg_ref[0] + agg_ref[1]
    t = (jnp.dot(x_ref[...], w1x_ref[...], preferred_element_type=jnp.float32)
         + jnp.dot(agg, w1a_ref[...], preferred_element_type=jnp.float32)
         + n1_ref[...])
    t = _silu(t)
    xn = x_ref[...] + jnp.dot(
        t, w2_ref[...], preferred_element_type=jnp.float32) + n2_ref[...]
    x_out[...] = xn
    a_out[...] = jnp.dot(xn, ws_ref[...],
                         preferred_element_type=jnp.float32) + e1_ref[...]
    b_out[...] = jnp.dot(xn, wd_ref[...], preferred_element_type=jnp.float32)


def _tc_node(x, agg2, nw1x, nw1a, nb1_i, nw2_i, nb2_i, wsrc, wdst, eb1_n):
    return pl.pallas_call(
        _node_body,
        grid=(N // BN,),
        in_specs=[
            _rows(BN, H),
            pl.BlockSpec((NC, BN, H), lambda n: (0, n, 0)),
            _full((H, H)), _full((H, H)), _full((1, H)),
            _full((H, H)), _full((1, H)),
            _full((H, H)), _full((H, H)), _full((1, H)),
        ],
        out_specs=[_rows(BN, H)] * 3,
        out_shape=[jax.ShapeDtypeStruct((N, H), jnp.float32)] * 3,
    )(x, agg2, nw1x, nw1a, nb1_i.reshape(1, H), nw2_i, nb2_i.reshape(1, H),
      wsrc, wdst, eb1_n.reshape(1, H))


def _node_final_body(x_ref, agg_ref, w1x_ref, w1a_ref, n1_ref, w2_ref,
                     n2_ref, wo_ref, bo_ref, o_ref):
    agg = agg_ref[0] + agg_ref[1]
    t = (jnp.dot(x_ref[...], w1x_ref[...], preferred_element_type=jnp.float32)
         + jnp.dot(agg, w1a_ref[...], preferred_element_type=jnp.float32)
         + n1_ref[...])
    t = _silu(t)
    xn = x_ref[...] + jnp.dot(
        t, w2_ref[...], preferred_element_type=jnp.float32) + n2_ref[...]
    o_ref[...] = jnp.dot(xn, wo_ref[...],
                         preferred_element_type=jnp.float32) + bo_ref[...]


def _tc_node_final(x, agg2, nw1x, nw1a, nb1_i, nw2_i, nb2_i, out_w, out_b):
    return pl.pallas_call(
        _node_final_body,
        grid=(N // BN,),
        in_specs=[
            _rows(BN, H),
            pl.BlockSpec((NC, BN, H), lambda n: (0, n, 0)),
            _full((H, H)), _full((H, H)), _full((1, H)),
            _full((H, H)), _full((1, H)),
            _full((H, H)), _full((1, H)),
        ],
        out_specs=_rows(BN, H),
        out_shape=jax.ShapeDtypeStruct((N, H), jnp.float32),
    )(x, agg2, nw1x, nw1a, nb1_i.reshape(1, H), nw2_i, nb2_i.reshape(1, H),
      out_w, out_b.reshape(1, H))


def kernel(h, edges, edge_attr, emb_w, emb_b, out_w, out_b,
           ew1, eb1, ew2, eb2, nw1, nb1, nw2, nb2):
    idx_halves = edges.reshape(2, 2, EH)
    idx1 = idx_halves[:, 0].reshape(2, NW, PT)
    idx2 = idx_halves[:, 1].reshape(2, NW, PT)
    row1 = idx1[0].reshape(NW, NCH, CH)
    row2 = idx2[0].reshape(NW, NCH, CH)
    ea1 = edge_attr[:EH]
    ea2 = edge_attr[EH:]
    zeros2 = jnp.zeros((NC, N_PAD, H), jnp.float32)

    x, a, b = _tc_embed(h, emb_w, emb_b, ew1[0, :H], ew1[0, H:2 * H], eb1[0])
    for i in range(L):
        g1 = _sc_gather(idx1, a, b)
        g2 = _sc_gather(idx2, a, b)
        m21 = _tc_edge(g1, ea1, ew1[i, 2 * H:], eb2[i], ew2[i])
        m22 = _tc_edge(g2, ea2, ew1[i, 2 * H:], eb2[i], ew2[i])
        s1 = _sc_scatter(row1, m21, zeros2)
        agg2 = _sc_scatter(row2, m22, s1)
        if i + 1 < L:
            x, a, b = _tc_node(x, agg2, nw1[i, :H], nw1[i, H:], nb1[i],
                               nw2[i], nb2[i], ew1[i + 1, :H],
                               ew1[i + 1, H:2 * H], eb1[i + 1])
        else:
            out = _tc_node_final(x, agg2, nw1[i, :H], nw1[i, H:], nb1[i],
                                 nw2[i], nb2[i], out_w, out_b)
    return out
